# baseline (speedup 1.0000x reference)
import jax
import jax.numpy as jnp
from jax.experimental import pallas as pl


def _fused_kernel(a_ref, b_ref, x_ref, ao_ref, bo_ref):
    a = a_ref[...]
    b = b_ref[...]
    ao_ref[...] = a
    bo_ref[...] = b
    x_ref[...] = jnp.sum(a * b, axis=1, keepdims=True)


def kernel(gum, gim):
    n, d = gum.shape
    blk = 512
    grid = n // blk
    x2d, a_out, b_out = pl.pallas_call(
        _fused_kernel,
        grid=(grid,),
        in_specs=[
            pl.BlockSpec((blk, d), lambda i: (i, 0)),
            pl.BlockSpec((blk, d), lambda i: (i, 0)),
        ],
        out_specs=(
            pl.BlockSpec((blk, 1), lambda i: (i, 0)),
            pl.BlockSpec((blk, d), lambda i: (i, 0)),
            pl.BlockSpec((blk, d), lambda i: (i, 0)),
        ),
        out_shape=(
            jax.ShapeDtypeStruct((n, 1), jnp.float32),
            jax.ShapeDtypeStruct((n, d), jnp.float32),
            jax.ShapeDtypeStruct((n, d), jnp.float32),
        ),
    )(gum, gim)
    return (x2d.reshape(n), a_out, b_out)


# speedup vs baseline: 1.5942x; 1.5942x over previous
import jax
import jax.numpy as jnp
from jax.experimental import pallas as pl


def _dot_kernel(a_ref, b_ref, x_ref):
    x_ref[...] = jnp.sum(a_ref[...] * b_ref[...], axis=1)


def kernel(gum, gim):
    n, d = gum.shape
    xui = pl.pallas_call(
        _dot_kernel,
        out_shape=jax.ShapeDtypeStruct((n,), jnp.float32),
    )(gum, gim)
    return (xui, gum, gim)


# speedup vs baseline: 5.5303x; 3.4690x over previous
import jax
import jax.numpy as jnp
from jax.experimental import pallas as pl


def _fused_t_kernel(at_ref, bt_ref, x_ref, ao_ref, bo_ref):
    a = at_ref[...]
    b = bt_ref[...]
    ao_ref[...] = a
    bo_ref[...] = b
    x_ref[...] = jnp.sum(a * b, axis=0)


def kernel(gum, gim):
    n, d = gum.shape
    at = gum.T
    bt = gim.T
    x, aot, bot = pl.pallas_call(
        _fused_t_kernel,
        out_shape=(
            jax.ShapeDtypeStruct((n,), jnp.float32),
            jax.ShapeDtypeStruct((d, n), jnp.float32),
            jax.ShapeDtypeStruct((d, n), jnp.float32),
        ),
    )(at, bt)
    return (x, aot.T, bot.T)
